# baseline (speedup 1.0000x reference)
import jax
import jax.numpy as jnp
from jax import lax
from jax.experimental import pallas as pl
from jax.experimental.pallas import tpu as pltpu


def _head_kernel(x_ref, w_ref, b_ref, o_ref):
    tb, T, H = x_ref.shape
    xf = x_ref[...].reshape(tb * T, H)
    z = lax.dot_general(w_ref[...], xf, (((1,), (1,)), ((), ())),
                        preferred_element_type=jnp.float32)
    z = z + b_ref[0]
    p = pl.reciprocal(1.0 + jnp.exp(-z), approx=True)
    o_ref[...] = p.reshape(tb, T).astype(o_ref.dtype)


@jax.jit
def _decision_head_fast(x, weight, bias):
    B, T, H = x.shape
    w = weight.reshape(1, H).astype(x.dtype)
    b1 = bias.reshape((1,)).astype(jnp.float32)

    tb = 64
    return pl.pallas_call(
        _head_kernel,
        out_shape=jax.ShapeDtypeStruct((B, T), x.dtype),
        grid=(pl.cdiv(B, tb),),
        in_specs=[
            pl.BlockSpec((tb, T, H), lambda i: (i, 0, 0)),
            pl.BlockSpec((1, H), lambda i: (0, 0)),
            pl.BlockSpec(memory_space=pltpu.MemorySpace.SMEM),
        ],
        out_specs=pl.BlockSpec((tb, T), lambda i: (i, 0)),
        compiler_params=pltpu.CompilerParams(
            dimension_semantics=("arbitrary",),
            vmem_limit_bytes=56 * 1024 * 1024,
        ),
    )(x, w, b1)


def kernel(x, weight, bias):
    return _decision_head_fast(x, weight, bias)


# speedup vs baseline: 1.0094x; 1.0094x over previous
import jax
import jax.numpy as jnp
from jax import lax
from jax.experimental import pallas as pl
from jax.experimental.pallas import tpu as pltpu

_NSTREAM = 4


def _head_kernel(x0_ref, x1_ref, x2_ref, x3_ref, w_ref, b_ref, o_ref):
    for k, x_ref in enumerate((x0_ref, x1_ref, x2_ref, x3_ref)):
        tbq, T, H = x_ref.shape
        xf = x_ref[...].reshape(tbq * T, H)
        z = lax.dot_general(w_ref[...], xf, (((1,), (1,)), ((), ())),
                            preferred_element_type=jnp.float32)
        z = z + b_ref[0]
        p = pl.reciprocal(1.0 + jnp.exp(-z), approx=True)
        o_ref[pl.ds(k * tbq, tbq), :] = p.reshape(tbq, T).astype(o_ref.dtype)


@jax.jit
def _decision_head_fast(x, weight, bias):
    B, T, H = x.shape
    w = weight.reshape(1, H).astype(x.dtype)
    b1 = bias.reshape((1,)).astype(jnp.float32)

    tbq = 8
    tb = tbq * _NSTREAM
    x_specs = [
        pl.BlockSpec((tbq, T, H), lambda i, k=k: (_NSTREAM * i + k, 0, 0))
        for k in range(_NSTREAM)
    ]
    return pl.pallas_call(
        _head_kernel,
        out_shape=jax.ShapeDtypeStruct((B, T), x.dtype),
        grid=(pl.cdiv(B, tb),),
        in_specs=x_specs + [
            pl.BlockSpec((1, H), lambda i: (0, 0)),
            pl.BlockSpec(memory_space=pltpu.MemorySpace.SMEM),
        ],
        out_specs=pl.BlockSpec((tb, T), lambda i: (i, 0)),
        compiler_params=pltpu.CompilerParams(
            dimension_semantics=("arbitrary",),
            vmem_limit_bytes=56 * 1024 * 1024,
        ),
    )(x, x, x, x, w, b1)


def kernel(x, weight, bias):
    return _decision_head_fast(x, weight, bias)


# speedup vs baseline: 1.1596x; 1.1488x over previous
import jax
import jax.numpy as jnp
from jax.experimental import pallas as pl
from jax.experimental.pallas import tpu as pltpu


def _head_kernel(y_ref, w_ref, b_ref, o_ref):
    tb, T = o_ref.shape
    z = jnp.dot(w_ref[...], y_ref[...], preferred_element_type=jnp.float32)
    z = z + b_ref[0]
    p = pl.reciprocal(1.0 + jnp.exp(-z), approx=True)
    o_ref[...] = p.reshape(tb, T).astype(o_ref.dtype)


@jax.jit
def _decision_head_fast(x, weight, bias):
    B, T, H = x.shape
    w = weight.reshape(1, H).astype(x.dtype)
    b1 = bias.reshape((1,)).astype(jnp.float32)

    y = x.reshape(B * T, H).T
    tb = 16
    nB = tb * T
    return pl.pallas_call(
        _head_kernel,
        out_shape=jax.ShapeDtypeStruct((B, T), x.dtype),
        grid=(pl.cdiv(B, tb),),
        in_specs=[
            pl.BlockSpec((H, nB), lambda i: (0, i)),
            pl.BlockSpec((1, H), lambda i: (0, 0)),
            pl.BlockSpec(memory_space=pltpu.MemorySpace.SMEM),
        ],
        out_specs=pl.BlockSpec((tb, T), lambda i: (i, 0)),
        compiler_params=pltpu.CompilerParams(
            dimension_semantics=("arbitrary",),
            vmem_limit_bytes=56 * 1024 * 1024,
        ),
    )(y, w, b1)


def kernel(x, weight, bias):
    return _decision_head_fast(x, weight, bias)
